# baseline (speedup 1.0000x reference)
import functools

import jax
import jax.numpy as jnp
from jax.experimental import pallas as pl

_NUM_E = 1024
_DIM = 64
_COMMITMENT_COST = 0.25


def _vq_block(x_ref, xn2b_ref, eb_ref, e2_ref, latent_ref, loss_ref):
    x = x_ref[...]
    eb = eb_ref[...]
    e2 = e2_ref[...]

    m2 = jax.lax.dot_general(
        xn2b_ref[...], eb, (((1,), (1,)), ((), ())),
        preferred_element_type=jnp.float32)
    x2 = jnp.sum(x * x, axis=1, keepdims=True)
    d = (x2 + e2) + m2

    dmin = jnp.min(d, axis=1, keepdims=True)
    col = jax.lax.broadcasted_iota(jnp.int32, (1, _NUM_E), 1)
    s = jnp.where(d == dmin, col, _NUM_E)
    idx = jnp.min(s, axis=1, keepdims=True)
    onehot = (s == idx).astype(jnp.bfloat16)
    e = jax.lax.dot_general(
        onehot, eb, (((1,), (0,)), ((), ())),
        preferred_element_type=jnp.float32)

    latent_ref[...] = x + (e - x)
    part = jnp.sum((e - x) ** 2, keepdims=True).reshape(1, 1)

    @pl.when(pl.program_id(0) == 0)
    def _():
        loss_ref[...] = jnp.zeros_like(loss_ref)
    loss_ref[...] += part


@functools.partial(jax.jit, static_argnames=("block_rows",))
def _vq(inputs, embeddings, block_rows=2048):
    x = inputs.reshape(-1, _DIM)
    n = x.shape[0]
    xn2b = (-2.0 * x).astype(jnp.bfloat16)
    eb = embeddings.astype(jnp.bfloat16)
    e2 = jnp.sum(embeddings ** 2, axis=1)[None, :]
    grid = (n // block_rows,)
    latent, loss_sum = pl.pallas_call(
        _vq_block,
        grid=grid,
        in_specs=[
            pl.BlockSpec((block_rows, _DIM), lambda i: (i, 0)),
            pl.BlockSpec((block_rows, _DIM), lambda i: (i, 0)),
            pl.BlockSpec((_NUM_E, _DIM), lambda i: (0, 0)),
            pl.BlockSpec((1, _NUM_E), lambda i: (0, 0)),
        ],
        out_specs=[
            pl.BlockSpec((block_rows, _DIM), lambda i: (i, 0)),
            pl.BlockSpec((1, 1), lambda i: (0, 0)),
        ],
        out_shape=[
            jax.ShapeDtypeStruct((n, _DIM), jnp.float32),
            jax.ShapeDtypeStruct((1, 1), jnp.float32),
        ],
    )(x, xn2b, eb, e2)
    mean_sq = loss_sum[0, 0] / jnp.float32(n * _DIM)
    loss = _COMMITMENT_COST * mean_sq + mean_sq
    return loss, latent.reshape(inputs.shape)


def kernel(inputs, embeddings):
    return _vq(inputs, embeddings)


# speedup vs baseline: 1.0765x; 1.0765x over previous
import functools

import jax
import jax.numpy as jnp
from jax.experimental import pallas as pl
from jax.experimental.pallas import tpu as pltpu

_NUM_E = 1024
_DIM = 64
_COMMITMENT_COST = 0.25


def _vq_block(x_ref, emb_ref, latent_ref, loss_ref, eb_ref, e2_ref):
    @pl.when(pl.program_id(0) == 0)
    def _():
        emb = emb_ref[...]
        eb_ref[...] = emb.astype(jnp.bfloat16)
        e2_ref[...] = jnp.sum(emb * emb, axis=1)[None, :]
        loss_ref[...] = jnp.zeros_like(loss_ref)

    x = x_ref[...]
    eb = eb_ref[...]
    e2 = e2_ref[...]

    m2 = jax.lax.dot_general(
        (-2.0 * x).astype(jnp.bfloat16), eb, (((1,), (1,)), ((), ())),
        preferred_element_type=jnp.float32)
    x2 = jnp.sum(x * x, axis=1, keepdims=True)
    d = (x2 + e2) + m2

    dmin = jnp.min(d, axis=1, keepdims=True)
    col = jax.lax.broadcasted_iota(jnp.int32, (1, _NUM_E), 1)
    s = jnp.where(d == dmin, col, _NUM_E)
    idx = jnp.min(s, axis=1, keepdims=True)
    onehot = (s == idx).astype(jnp.bfloat16)
    e = jax.lax.dot_general(
        onehot, eb, (((1,), (0,)), ((), ())),
        preferred_element_type=jnp.float32)

    latent_ref[...] = x + (e - x)
    loss_ref[...] += jnp.sum((e - x) ** 2, keepdims=True).reshape(1, 1)


@functools.partial(jax.jit, static_argnames=("block_rows",))
def _vq(inputs, embeddings, block_rows=2048):
    x = inputs.reshape(-1, _DIM)
    n = x.shape[0]
    grid = (n // block_rows,)
    latent, loss_sum = pl.pallas_call(
        _vq_block,
        grid=grid,
        in_specs=[
            pl.BlockSpec((block_rows, _DIM), lambda i: (i, 0)),
            pl.BlockSpec((_NUM_E, _DIM), lambda i: (0, 0)),
        ],
        out_specs=[
            pl.BlockSpec((block_rows, _DIM), lambda i: (i, 0)),
            pl.BlockSpec((1, 1), lambda i: (0, 0)),
        ],
        out_shape=[
            jax.ShapeDtypeStruct((n, _DIM), jnp.float32),
            jax.ShapeDtypeStruct((1, 1), jnp.float32),
        ],
        scratch_shapes=[
            pltpu.VMEM((_NUM_E, _DIM), jnp.bfloat16),
            pltpu.VMEM((1, _NUM_E), jnp.float32),
        ],
    )(x, embeddings)
    mean_sq = loss_sum[0, 0] / jnp.float32(n * _DIM)
    loss = _COMMITMENT_COST * mean_sq + mean_sq
    return loss, latent.reshape(inputs.shape)


def kernel(inputs, embeddings):
    return _vq(inputs, embeddings)


# speedup vs baseline: 1.1050x; 1.0264x over previous
import functools

import jax
import jax.numpy as jnp
from jax.experimental import pallas as pl
from jax.experimental.pallas import tpu as pltpu

_NUM_E = 1024
_DIM = 64
_COMMITMENT_COST = 0.25


def _vq_block(x_ref, emb_ref, latent_ref, loss_ref, eb_ref, e2_ref):
    @pl.when(pl.program_id(0) == 0)
    def _():
        emb = emb_ref[...]
        eb_ref[...] = emb.astype(jnp.bfloat16)
        e2_ref[...] = jnp.sum(emb * emb, axis=1)[None, :]
        loss_ref[...] = jnp.zeros_like(loss_ref)

    x = x_ref[...]
    eb = eb_ref[...]
    e2 = e2_ref[...]

    m2 = jax.lax.dot_general(
        (-2.0 * x).astype(jnp.bfloat16), eb, (((1,), (1,)), ((), ())),
        preferred_element_type=jnp.float32)
    x2 = jnp.sum(x * x, axis=1, keepdims=True)
    d = (x2 + e2) + m2

    dmin = jnp.min(d, axis=1, keepdims=True)
    col = jax.lax.broadcasted_iota(jnp.int32, (1, _NUM_E), 1)
    s = jnp.where(d == dmin, col, _NUM_E)
    idx = jnp.min(s, axis=1, keepdims=True)
    onehot = (s == idx).astype(jnp.bfloat16)
    e = jax.lax.dot_general(
        onehot, eb, (((1,), (0,)), ((), ())),
        preferred_element_type=jnp.float32)

    latent_ref[...] = x + (e - x)
    loss_ref[...] += jnp.sum((e - x) ** 2, keepdims=True).reshape(1, 1)


@functools.partial(jax.jit, static_argnames=("block_rows",))
def _vq(inputs, embeddings, block_rows=4096):
    x = inputs.reshape(-1, _DIM)
    n = x.shape[0]
    grid = (n // block_rows,)
    latent, loss_sum = pl.pallas_call(
        _vq_block,
        grid=grid,
        in_specs=[
            pl.BlockSpec((block_rows, _DIM), lambda i: (i, 0)),
            pl.BlockSpec((_NUM_E, _DIM), lambda i: (0, 0)),
        ],
        out_specs=[
            pl.BlockSpec((block_rows, _DIM), lambda i: (i, 0)),
            pl.BlockSpec((1, 1), lambda i: (0, 0)),
        ],
        out_shape=[
            jax.ShapeDtypeStruct((n, _DIM), jnp.float32),
            jax.ShapeDtypeStruct((1, 1), jnp.float32),
        ],
        scratch_shapes=[
            pltpu.VMEM((_NUM_E, _DIM), jnp.bfloat16),
            pltpu.VMEM((1, _NUM_E), jnp.float32),
        ],
    )(x, embeddings)
    mean_sq = loss_sum[0, 0] / jnp.float32(n * _DIM)
    loss = _COMMITMENT_COST * mean_sq + mean_sq
    return loss, latent.reshape(inputs.shape)


def kernel(inputs, embeddings):
    return _vq(inputs, embeddings)
